# baseline (device time: 45331 ns/iter reference)
import jax
import jax.numpy as jnp
from jax import lax
from jax.experimental import pallas as pl
from jax.experimental.pallas import tpu as pltpu

N_DEV = 4
P = 64


def kernel(x):
    m, n = x.shape
    r = m // P
    w = P * n

    def body(x_hbm, out_hbm, z1, z2, comm_ref, in_sems, out_sems,
             send_sems, recv_sems):
        my = lax.axis_index("i")
        left = (my - 1) % N_DEV
        right = (my + 1) % N_DEV

        in_copies = []
        for p in range(P):
            c = pltpu.make_async_copy(
                x_hbm.at[pl.ds(p * r, r), :],
                z1.at[:, pl.ds(p * n, n)],
                in_sems.at[p],
            )
            c.start()
            in_copies.append(c)

        barrier_sem = pltpu.get_barrier_semaphore()
        for nbr in (left, right):
            pl.semaphore_signal(
                barrier_sem, inc=1,
                device_id=(nbr,), device_id_type=pl.DeviceIdType.MESH,
            )
        pl.semaphore_wait(barrier_sem, 2)
        for c in in_copies:
            c.wait()

        bufs = (z1, z2)
        for k, s in enumerate([1, 2, 4, 8]):
            src, dst = bufs[k % 2], bufs[(k + 1) % 2]
            dst[pl.ds(0, s), :] = src[pl.ds(0, s), :]
            dst[pl.ds(s, r - s), :] = (
                src[pl.ds(s, r - s), :] * src[pl.ds(0, r - s), :]
            )

        t = (z1[pl.ds(r // 4 - 1, 1), :] * z1[pl.ds(r // 2 - 1, 1), :]) * (
            z1[pl.ds(3 * r // 4 - 1, 1), :] * z1[pl.ds(r - 1, 1), :]
        )
        sb = 1
        while sb < P:
            t = t * jnp.concatenate(
                [jnp.ones((1, sb * n), jnp.float32), t[:, : w - sb * n]],
                axis=1,
            )
            sb *= 2
        excl = jnp.concatenate(
            [jnp.ones((1, n), jnp.float32), t[:, : w - n]], axis=1
        )
        comm_ref[0, :, :] = t[:, w - n:]

        def make_hop(h):
            return pltpu.make_async_remote_copy(
                src_ref=comm_ref.at[h % 2],
                dst_ref=comm_ref.at[(h + 1) % 2],
                send_sem=send_sems.at[h % 2],
                recv_sem=recv_sems.at[(h + 1) % 2],
                device_id=(right,),
                device_id_type=pl.DeviceIdType.MESH,
            )

        def fold(h, acc):
            origin = (my - h - 1) % N_DEV
            chunk = comm_ref[(h + 1) % 2, :, :]
            return acc * jnp.where(origin < my, chunk, jnp.float32(1.0))

        acc = jnp.ones((1, n), jnp.float32)
        hop0 = make_hop(0)
        hop0.start()

        z2[pl.ds(0, 16), :] = z1[pl.ds(0, 16), :]
        z2[pl.ds(16, r - 16), :] = (
            z1[pl.ds(16, r - 16), :] * z1[pl.ds(0, r - 16), :]
        )

        hop0.wait()
        acc = fold(0, acc)
        hop1 = make_hop(1)
        hop1.start()

        z1[pl.ds(0, 32), :] = z2[pl.ds(0, 32), :]
        z1[pl.ds(32, r - 32), :] = (
            z2[pl.ds(32, r - 32), :] * z2[pl.ds(0, r - 32), :]
        )

        hop1.wait()
        acc = fold(1, acc)
        hop2 = make_hop(2)
        hop2.start()
        hop2.wait()
        acc = fold(2, acc)

        bcast = excl * jnp.concatenate([acc] * P, axis=1)
        z2[:, :] = z1[:, :] * bcast

        out_copies = []
        for p in range(P):
            c = pltpu.make_async_copy(
                z2.at[:, pl.ds(p * n, n)],
                out_hbm.at[pl.ds(p * r, r), :],
                out_sems.at[p],
            )
            c.start()
            out_copies.append(c)
        for c in out_copies:
            c.wait()

    return pl.pallas_call(
        body,
        out_shape=jax.ShapeDtypeStruct((m, n), jnp.float32),
        in_specs=[pl.BlockSpec(memory_space=pl.ANY)],
        out_specs=pl.BlockSpec(memory_space=pl.ANY),
        scratch_shapes=[
            pltpu.VMEM((m // P, P * n), jnp.float32),
            pltpu.VMEM((m // P, P * n), jnp.float32),
            pltpu.VMEM((2, 1, n), jnp.float32),
            pltpu.SemaphoreType.DMA((P,)),
            pltpu.SemaphoreType.DMA((P,)),
            pltpu.SemaphoreType.DMA((2,)),
            pltpu.SemaphoreType.DMA((2,)),
        ],
        compiler_params=pltpu.CompilerParams(
            collective_id=0,
            vmem_limit_bytes=100 * 1024 * 1024,
        ),
    )(x)


# device time: 33937 ns/iter; 1.3357x vs baseline; 1.3357x over previous
import jax
import jax.numpy as jnp
from jax import lax
from jax.experimental import pallas as pl
from jax.experimental.pallas import tpu as pltpu

N_DEV = 4
P = 64


def kernel(x):
    m, n = x.shape
    r = m // P
    w = P * n

    def body(x_hbm, out_hbm, z1, z2, comm_ref, in_sems, out_sems,
             send_sems, recv_sems):
        my = lax.axis_index("i")
        left = (my - 1) % N_DEV
        right = (my + 1) % N_DEV

        in_copies = []
        for p in range(P):
            c = pltpu.make_async_copy(
                x_hbm.at[pl.ds(p * r, r), :],
                z1.at[:, pl.ds(p * n, n)],
                in_sems.at[p],
            )
            c.start()
            in_copies.append(c)

        barrier_sem = pltpu.get_barrier_semaphore()
        for nbr in (left, right):
            pl.semaphore_signal(
                barrier_sem, inc=1,
                device_id=(nbr,), device_id_type=pl.DeviceIdType.MESH,
            )
        pl.semaphore_wait(barrier_sem, 2)
        for c in in_copies:
            c.wait()

        bufs = (z1, z2)
        for k, s in enumerate([1, 2, 4, 8]):
            src, dst = bufs[k % 2], bufs[(k + 1) % 2]
            dst[pl.ds(0, s), :] = src[pl.ds(0, s), :]
            dst[pl.ds(s, r - s), :] = (
                src[pl.ds(s, r - s), :] * src[pl.ds(0, r - s), :]
            )

        t = (z1[pl.ds(r // 4 - 1, 1), :] * z1[pl.ds(r // 2 - 1, 1), :]) * (
            z1[pl.ds(3 * r // 4 - 1, 1), :] * z1[pl.ds(r - 1, 1), :]
        )
        sb = 1
        while sb < P:
            t = t * jnp.concatenate(
                [jnp.ones((1, sb * n), jnp.float32), t[:, : w - sb * n]],
                axis=1,
            )
            sb *= 2
        excl = jnp.concatenate(
            [jnp.ones((1, n), jnp.float32), t[:, : w - n]], axis=1
        )
        comm_ref[0, :, :] = t[:, w - n:]

        def make_hop(h):
            return pltpu.make_async_remote_copy(
                src_ref=comm_ref.at[h % 2],
                dst_ref=comm_ref.at[(h + 1) % 2],
                send_sem=send_sems.at[h % 2],
                recv_sem=recv_sems.at[(h + 1) % 2],
                device_id=(right,),
                device_id_type=pl.DeviceIdType.MESH,
            )

        def fold(h, acc):
            origin = (my - h - 1) % N_DEV
            chunk = comm_ref[(h + 1) % 2, :, :]
            return acc * jnp.where(origin < my, chunk, jnp.float32(1.0))

        acc = jnp.ones((1, n), jnp.float32)
        hop0 = make_hop(0)
        hop0.start()

        z2[pl.ds(0, 16), :] = z1[pl.ds(0, 16), :]
        z2[pl.ds(16, r - 16), :] = (
            z1[pl.ds(16, r - 16), :] * z1[pl.ds(0, r - 16), :]
        )

        hop0.wait()
        acc = fold(0, acc)
        hop1 = make_hop(1)
        hop1.start()

        z1[pl.ds(0, 32), :] = z2[pl.ds(0, 32), :]
        z1[pl.ds(32, r - 32), :] = (
            z2[pl.ds(32, r - 32), :] * z2[pl.ds(0, r - 32), :]
        )

        hop1.wait()
        acc = fold(1, acc)
        hop2 = make_hop(2)
        hop2.start()
        hop2.wait()
        acc = fold(2, acc)

        bcast = excl * jnp.concatenate([acc] * P, axis=1)
        z2[:, :] = z1[:, :] * bcast

        out_copies = []
        for p in range(P):
            c = pltpu.make_async_copy(
                z2.at[:, pl.ds(p * n, n)],
                out_hbm.at[pl.ds(p * r, r), :],
                out_sems.at[p],
            )
            c.start()
            out_copies.append(c)
        for c in out_copies:
            c.wait()

    return pl.pallas_call(
        body,
        out_shape=jax.ShapeDtypeStruct((m, n), jnp.float32),
        in_specs=[pl.BlockSpec(memory_space=pl.ANY)],
        out_specs=pl.BlockSpec(memory_space=pl.ANY),
        scratch_shapes=[
            pltpu.VMEM((m // P, P * n), jnp.float32),
            pltpu.VMEM((m // P, P * n), jnp.float32),
            pltpu.VMEM((2, 1, n), jnp.float32),
            pltpu.SemaphoreType.DMA((P,)),
            pltpu.SemaphoreType.DMA((P,)),
            pltpu.SemaphoreType.DMA((2,)),
            pltpu.SemaphoreType.DMA((2,)),
        ],
        compiler_params=pltpu.CompilerParams(
            collective_id=0,
            vmem_limit_bytes=36 * 1024 * 1024,
        ),
    )(x)


# device time: 33760 ns/iter; 1.3427x vs baseline; 1.0052x over previous
import jax
import jax.numpy as jnp
from jax import lax
from jax.experimental import pallas as pl
from jax.experimental.pallas import tpu as pltpu

N_DEV = 4
P = 64


def kernel(x):
    m, n = x.shape
    r = m // P
    w = P * n

    def body(x_hbm, out_hbm, z1, z2, comm_ref, in_sems, out_sems,
             send_sems, recv_sems):
        my = lax.axis_index("i")
        left = (my - 1) % N_DEV
        right = (my + 1) % N_DEV

        in_copies = []
        for p in range(P):
            c = pltpu.make_async_copy(
                x_hbm.at[pl.ds(p * r, r), :],
                z1.at[:, pl.ds(p * n, n)],
                in_sems.at[p],
            )
            c.start()
            in_copies.append(c)

        barrier_sem = pltpu.get_barrier_semaphore()
        for nbr in (left, right):
            pl.semaphore_signal(
                barrier_sem, inc=1,
                device_id=(nbr,), device_id_type=pl.DeviceIdType.MESH,
            )
        pl.semaphore_wait(barrier_sem, 2)
        for c in in_copies:
            c.wait()

        bufs = (z1, z2)
        for k, s in enumerate([1, 2, 4, 8]):
            src, dst = bufs[k % 2], bufs[(k + 1) % 2]
            dst[pl.ds(0, s), :] = src[pl.ds(0, s), :]
            dst[pl.ds(s, r - s), :] = (
                src[pl.ds(s, r - s), :] * src[pl.ds(0, r - s), :]
            )

        t = (z1[pl.ds(r // 4 - 1, 1), :] * z1[pl.ds(r // 2 - 1, 1), :]) * (
            z1[pl.ds(3 * r // 4 - 1, 1), :] * z1[pl.ds(r - 1, 1), :]
        )
        sb = 1
        while sb < P:
            t = t * jnp.concatenate(
                [jnp.ones((1, sb * n), jnp.float32), t[:, : w - sb * n]],
                axis=1,
            )
            sb *= 2
        excl = jnp.concatenate(
            [jnp.ones((1, n), jnp.float32), t[:, : w - n]], axis=1
        )
        comm_ref[0, :, :] = t[:, w - n:]

        def make_hop(h):
            return pltpu.make_async_remote_copy(
                src_ref=comm_ref.at[h % 2],
                dst_ref=comm_ref.at[(h + 1) % 2],
                send_sem=send_sems.at[h % 2],
                recv_sem=recv_sems.at[(h + 1) % 2],
                device_id=(right,),
                device_id_type=pl.DeviceIdType.MESH,
            )

        def fold(h, acc):
            origin = (my - h - 1) % N_DEV
            chunk = comm_ref[(h + 1) % 2, :, :]
            return acc * jnp.where(origin < my, chunk, jnp.float32(1.0))

        acc = jnp.ones((1, n), jnp.float32)
        hop0 = make_hop(0)
        hop0.start()

        z2[pl.ds(0, 16), :] = z1[pl.ds(0, 16), :]
        z2[pl.ds(16, r - 16), :] = (
            z1[pl.ds(16, r - 16), :] * z1[pl.ds(0, r - 16), :]
        )

        hop0.wait()
        acc = fold(0, acc)
        hop1 = make_hop(1)
        hop1.start()

        z1[pl.ds(0, 32), :] = z2[pl.ds(0, 32), :]
        z1[pl.ds(32, r - 32), :] = (
            z2[pl.ds(32, r - 32), :] * z2[pl.ds(0, r - 32), :]
        )

        hop1.wait()
        acc = fold(1, acc)
        hop2 = make_hop(2)
        hop2.start()
        hop2.wait()
        acc = fold(2, acc)

        bcast = excl * jnp.concatenate([acc] * P, axis=1)
        n_chunks = 4
        pc = P // n_chunks
        wc = w // n_chunks
        out_copies = []
        for q in range(n_chunks):
            z2[:, pl.ds(q * wc, wc)] = (
                z1[:, pl.ds(q * wc, wc)] * bcast[:, q * wc: (q + 1) * wc]
            )
            for p in range(q * pc, (q + 1) * pc):
                c = pltpu.make_async_copy(
                    z2.at[:, pl.ds(p * n, n)],
                    out_hbm.at[pl.ds(p * r, r), :],
                    out_sems.at[p],
                )
                c.start()
                out_copies.append(c)
        for c in out_copies:
            c.wait()

    return pl.pallas_call(
        body,
        out_shape=jax.ShapeDtypeStruct((m, n), jnp.float32),
        in_specs=[pl.BlockSpec(memory_space=pl.ANY)],
        out_specs=pl.BlockSpec(memory_space=pl.ANY),
        scratch_shapes=[
            pltpu.VMEM((m // P, P * n), jnp.float32),
            pltpu.VMEM((m // P, P * n), jnp.float32),
            pltpu.VMEM((2, 1, n), jnp.float32),
            pltpu.SemaphoreType.DMA((P,)),
            pltpu.SemaphoreType.DMA((P,)),
            pltpu.SemaphoreType.DMA((2,)),
            pltpu.SemaphoreType.DMA((2,)),
        ],
        compiler_params=pltpu.CompilerParams(
            collective_id=0,
            vmem_limit_bytes=36 * 1024 * 1024,
        ),
    )(x)
